# baseline (speedup 1.0000x reference)
import functools

import jax
import jax.numpy as jnp
from jax import lax
from jax.experimental import pallas as pl
from jax.experimental.pallas import tpu as pltpu
from jax.experimental.pallas import tpu_sc as plsc

N_TOK = 32768
DIM = 64
K_CODES = 1024
TILE = 1024
GRID = N_TOK // TILE
LOSS_SCALE = 1.25 / (N_TOK * DIM)

SUBT = 512
CSUB = 8
N_CGROUP = K_CODES // CSUB


def _tc_dist_argmin(zt_ref, emb_ref, zsq_ref, esq_ref, idx_ref, loss_ref):
    emb = emb_ref[...]
    esqc = esq_ref[...]
    zm2t = -2.0 * zt_ref[...]
    sub = lax.broadcasted_iota(jnp.int32, (CSUB, SUBT), 0)
    loss_acc = jnp.zeros((1, 1), jnp.float32)
    for r in range(TILE // SUBT):
        dot2t = lax.dot_general(emb, zm2t[:, r * SUBT:(r + 1) * SUBT],
                                (((1,), (0,)), ((), ())),
                                preferred_element_type=jnp.float32)
        zst = zsq_ref[:, r * SUBT:(r + 1) * SUBT]
        m = (zst + esqc[0:CSUB, :]) + dot2t[0:CSUB, :]
        c1 = jnp.zeros((CSUB, SUBT), jnp.int32)
        for c in range(1, N_CGROUP):
            dd = (zst + esqc[c * CSUB:(c + 1) * CSUB, :]) \
                + dot2t[c * CSUB:(c + 1) * CSUB, :]
            pred = dd < m
            m = jnp.where(pred, dd, m)
            c1 = jnp.where(pred, jnp.int32(c), c1)
        gmin = jnp.min(m, axis=0, keepdims=True)
        j8 = c1 * CSUB + sub
        idx_ref[0, 0, r * SUBT:(r + 1) * SUBT] = jnp.min(
            jnp.where(m == gmin, j8, jnp.int32(K_CODES)),
            axis=0)
        loss_acc += jnp.sum(gmin).reshape(1, 1)

    @pl.when(pl.program_id(0) == 0)
    def _init():
        loss_ref[...] = jnp.zeros((1, 1), jnp.float32)

    loss_ref[...] += loss_acc

    @pl.when(pl.program_id(0) == GRID - 1)
    def _finish():
        loss_ref[...] = loss_ref[...] * jnp.float32(LOSS_SCALE)


def _sc_gather(embedding, indices):
    info = plsc.get_sparse_core_info()
    nc, ns = info.num_cores, info.num_subcores
    nw = nc * ns
    b_per_w = N_TOK // nw
    mesh = plsc.VectorSubcoreMesh(core_axis_name="c", subcore_axis_name="s")

    @functools.partial(
        pl.kernel,
        out_type=jax.ShapeDtypeStruct((N_TOK, DIM), jnp.float32),
        mesh=mesh,
        scratch_types=[
            pltpu.VMEM((b_per_w,), jnp.int32),
            pltpu.VMEM((b_per_w, DIM), jnp.float32),
            pltpu.SemaphoreType.DMA,
        ],
        compiler_params=pltpu.CompilerParams(use_tc_tiling_on_sc=False),
    )
    def gather_k(table_hbm, idx_hbm, out_hbm, idx_v, rows_v, sem):
        wid = lax.axis_index("s") * nc + lax.axis_index("c")
        base = wid * b_per_w
        pltpu.sync_copy(idx_hbm.at[pl.ds(base, b_per_w)], idx_v)
        pltpu.async_copy(table_hbm.at[idx_v], rows_v, sem).wait()
        pltpu.sync_copy(rows_v, out_hbm.at[pl.ds(base, b_per_w)])

    return gather_k(embedding, indices)


def kernel(z, embedding):
    z_sq = jnp.sum(z ** 2, axis=1).reshape(1, N_TOK)
    emb_sq = jnp.sum(embedding ** 2, axis=1).reshape(K_CODES, 1)
    idx3d, loss2d = pl.pallas_call(
        _tc_dist_argmin,
        grid=(GRID,),
        in_specs=[
            pl.BlockSpec((DIM, TILE), lambda i: (0, i)),
            pl.BlockSpec((K_CODES, DIM), lambda i: (0, 0)),
            pl.BlockSpec((1, TILE), lambda i: (0, i)),
            pl.BlockSpec((K_CODES, 1), lambda i: (0, 0)),
        ],
        out_specs=[
            pl.BlockSpec((1, 1, TILE), lambda i: (i, 0, 0)),
            pl.BlockSpec((1, 1), lambda i: (0, 0)),
        ],
        out_shape=[
            jax.ShapeDtypeStruct((GRID, 1, TILE), jnp.int32),
            jax.ShapeDtypeStruct((1, 1), jnp.float32),
        ],
    )(z.T, embedding, z_sq, emb_sq)
    indices = idx3d.reshape(N_TOK)
    z_q = _sc_gather(embedding, indices)
    loss = loss2d[0, 0]
    return (z_q, loss, indices)


# speedup vs baseline: 1.0037x; 1.0037x over previous
import functools

import jax
import jax.numpy as jnp
from jax import lax
from jax.experimental import pallas as pl
from jax.experimental.pallas import tpu as pltpu
from jax.experimental.pallas import tpu_sc as plsc

N_TOK = 32768
DIM = 64
K_CODES = 1024
TILE = 1024
GRID = N_TOK // TILE
LOSS_SCALE = 1.25 / (N_TOK * DIM)

SUBT = 256
CSUB = 8
N_CGROUP = K_CODES // CSUB


def _tc_dist_argmin(zt_ref, emb_ref, zsq_ref, esq_ref, idx_ref, loss_ref):
    emb = emb_ref[...]
    esqc = esq_ref[...]
    zm2t = -2.0 * zt_ref[...]
    sub = lax.broadcasted_iota(jnp.int32, (CSUB, SUBT), 0)
    loss_acc = jnp.zeros((1, 1), jnp.float32)
    for r in range(TILE // SUBT):
        dot2t = lax.dot_general(emb, zm2t[:, r * SUBT:(r + 1) * SUBT],
                                (((1,), (0,)), ((), ())),
                                preferred_element_type=jnp.float32)
        zst = zsq_ref[:, r * SUBT:(r + 1) * SUBT]
        m = (zst + esqc[0:CSUB, :]) + dot2t[0:CSUB, :]
        c1 = jnp.zeros((CSUB, SUBT), jnp.int32)
        for c in range(1, N_CGROUP):
            dd = (zst + esqc[c * CSUB:(c + 1) * CSUB, :]) \
                + dot2t[c * CSUB:(c + 1) * CSUB, :]
            pred = dd < m
            m = jnp.where(pred, dd, m)
            c1 = jnp.where(pred, jnp.int32(c), c1)
        gmin = jnp.min(m, axis=0, keepdims=True)
        j8 = c1 * CSUB + sub
        idx_ref[0, 0, r * SUBT:(r + 1) * SUBT] = jnp.min(
            jnp.where(m == gmin, j8, jnp.int32(K_CODES)),
            axis=0)
        loss_acc += jnp.sum(gmin).reshape(1, 1)

    @pl.when(pl.program_id(0) == 0)
    def _init():
        loss_ref[...] = jnp.zeros((1, 1), jnp.float32)

    loss_ref[...] += loss_acc

    @pl.when(pl.program_id(0) == GRID - 1)
    def _finish():
        loss_ref[...] = loss_ref[...] * jnp.float32(LOSS_SCALE)


def _sc_gather(embedding, indices):
    info = plsc.get_sparse_core_info()
    nc, ns = info.num_cores, info.num_subcores
    nw = nc * ns
    b_per_w = N_TOK // nw
    mesh = plsc.VectorSubcoreMesh(core_axis_name="c", subcore_axis_name="s")

    @functools.partial(
        pl.kernel,
        out_type=jax.ShapeDtypeStruct((N_TOK, DIM), jnp.float32),
        mesh=mesh,
        scratch_types=[
            pltpu.VMEM((b_per_w,), jnp.int32),
            pltpu.VMEM((b_per_w, DIM), jnp.float32),
            pltpu.SemaphoreType.DMA,
        ],
        compiler_params=pltpu.CompilerParams(use_tc_tiling_on_sc=False),
    )
    def gather_k(table_hbm, idx_hbm, out_hbm, idx_v, rows_v, sem):
        wid = lax.axis_index("s") * nc + lax.axis_index("c")
        base = wid * b_per_w
        pltpu.sync_copy(idx_hbm.at[pl.ds(base, b_per_w)], idx_v)
        pltpu.async_copy(table_hbm.at[idx_v], rows_v, sem).wait()
        pltpu.sync_copy(rows_v, out_hbm.at[pl.ds(base, b_per_w)])

    return gather_k(embedding, indices)


def kernel(z, embedding):
    z_sq = jnp.sum(z ** 2, axis=1).reshape(1, N_TOK)
    emb_sq = jnp.sum(embedding ** 2, axis=1).reshape(K_CODES, 1)
    idx3d, loss2d = pl.pallas_call(
        _tc_dist_argmin,
        grid=(GRID,),
        in_specs=[
            pl.BlockSpec((DIM, TILE), lambda i: (0, i)),
            pl.BlockSpec((K_CODES, DIM), lambda i: (0, 0)),
            pl.BlockSpec((1, TILE), lambda i: (0, i)),
            pl.BlockSpec((K_CODES, 1), lambda i: (0, 0)),
        ],
        out_specs=[
            pl.BlockSpec((1, 1, TILE), lambda i: (i, 0, 0)),
            pl.BlockSpec((1, 1), lambda i: (0, 0)),
        ],
        out_shape=[
            jax.ShapeDtypeStruct((GRID, 1, TILE), jnp.int32),
            jax.ShapeDtypeStruct((1, 1), jnp.float32),
        ],
    )(z.T, embedding, z_sq, emb_sq)
    indices = idx3d.reshape(N_TOK)
    z_q = _sc_gather(embedding, indices)
    loss = loss2d[0, 0]
    return (z_q, loss, indices)
